# baseline (speedup 1.0000x reference)
import functools

import jax
import jax.numpy as jnp
from jax import lax
from jax.experimental import pallas as pl
from jax.experimental.pallas import tpu as pltpu, tpu_sc as plsc

_PAD_C = 16
_ROW_BLK = 4096


def _proj_body(e_ref, w_ref, b_ref, o_ref):
    _, r8 = e_ref.shape
    q = lax.dot_general(
        e_ref[...], w_ref[...],
        (((0,), (0,)), ((), ())),
        preferred_element_type=jnp.float32,
    )
    i0 = lax.broadcasted_iota(jnp.int32, q.shape, 0)
    i1 = lax.broadcasted_iota(jnp.int32, q.shape, 1)
    qm = jnp.where((i0 % 8) == (i1 // _PAD_C), q, 0.0)
    o_ref[...] = qm.reshape(r8 // 8, 8, 128).sum(axis=1) + b_ref[...]


def _project_table(emb_t, w_tiled, b_tiled):
    e, v = emb_t.shape
    grid = (v + _ROW_BLK - 1) // _ROW_BLK
    return pl.pallas_call(
        _proj_body,
        grid=(grid,),
        in_specs=[
            pl.BlockSpec((e, _ROW_BLK), lambda i: (0, i)),
            pl.BlockSpec((e, 128), lambda i: (0, 0)),
            pl.BlockSpec((1, 128), lambda i: (0, 0)),
        ],
        out_specs=pl.BlockSpec((_ROW_BLK // 8, 128), lambda i: (i, 0)),
        out_shape=jax.ShapeDtypeStruct((v // 8, 128), jnp.float32),
    )(emb_t, w_tiled, b_tiled)


def _make_sc_pool(n_batch, seq_len, n_workers):
    per_w_tok = n_batch * seq_len // n_workers
    per_w_b = n_batch // n_workers
    chunk = 128
    n_chunks = per_w_tok // chunk
    fire = 10
    mesh = plsc.VectorSubcoreMesh(core_axis_name="c", subcore_axis_name="s")

    @functools.partial(
        pl.kernel,
        out_type=jax.ShapeDtypeStruct((n_batch, _PAD_C), jnp.float32),
        mesh=mesh,
        scratch_types=[
            pltpu.VMEM((n_chunks, chunk), jnp.int32),
            pltpu.VMEM((per_w_tok, _PAD_C), jnp.float32),
            pltpu.VMEM((per_w_b, _PAD_C), jnp.float32),
            pltpu.SemaphoreType.DMA,
        ],
        compiler_params=pltpu.CompilerParams(use_tc_tiling_on_sc=False),
    )
    def sc_pool(idx_hbm, p_hbm, out_hbm, idx_v, rows_v, out_v, sem):
        nc = mesh.num_cores
        wid = lax.axis_index("s") * nc + lax.axis_index("c")

        pltpu.sync_copy(idx_hbm.at[wid], idx_v)

        def gather_group(g, _):
            base = g * fire
            copies = []
            for i in range(fire):
                j = base + i
                copies.append(
                    pltpu.async_copy(
                        p_hbm.at[idx_v.at[j]],
                        rows_v.at[pl.ds(j * chunk, chunk)],
                        sem,
                    )
                )
            for c in copies:
                c.wait()
            return _

        lax.fori_loop(0, n_chunks // fire, gather_group, None)

        def pool_one(b, _):
            base = b * seq_len
            acc0 = rows_v[base]
            acc1 = rows_v[base + 1]
            for t in range(2, seq_len, 2):
                acc0 = acc0 + rows_v[base + t]
                acc1 = acc1 + rows_v[base + t + 1]
            out_v[b] = acc0 + acc1
            return _

        lax.fori_loop(0, per_w_b, pool_one, None)

        pltpu.sync_copy(out_v, out_hbm.at[pl.ds(wid * per_w_b, per_w_b)])

    return sc_pool


def kernel(text, emb_table, fc_w, fc_b):
    n_batch, seq_len = text.shape
    v, e = emb_table.shape
    c = fc_w.shape[1]
    info = plsc.get_sparse_core_info()
    n_workers = info.num_cores * info.num_subcores

    inv_l = 1.0 / seq_len
    w_pad = jnp.zeros((e, _PAD_C), jnp.float32).at[:, :c].set(fc_w * inv_l)
    b_pad = jnp.zeros((1, _PAD_C), jnp.float32).at[0, :c].set(fc_b * inv_l)
    w_tiled = jnp.tile(w_pad, (1, 128 // _PAD_C))
    b_tiled = jnp.tile(b_pad, (1, 128 // _PAD_C))
    proj = _project_table(emb_table.T, w_tiled, b_tiled)
    proj16 = proj.reshape(v, _PAD_C)

    idx = text.astype(jnp.int32).reshape(n_workers, -1, 128)
    pooled = _make_sc_pool(n_batch, seq_len, n_workers)(idx, proj16)
    return pooled[:, :c]


# speedup vs baseline: 1.0737x; 1.0737x over previous
import functools

import jax
import jax.numpy as jnp
from jax import lax
from jax.experimental import pallas as pl
from jax.experimental.pallas import tpu as pltpu, tpu_sc as plsc

_PAD_C = 16
_ROW_BLK = 8192


def _proj_body(e_ref, w_ref, b_ref, o_ref):
    _, r8 = e_ref.shape
    q = lax.dot_general(
        e_ref[...], w_ref[...],
        (((0,), (0,)), ((), ())),
        preferred_element_type=jnp.float32,
    )
    i0 = lax.broadcasted_iota(jnp.int32, q.shape, 0)
    i1 = lax.broadcasted_iota(jnp.int32, q.shape, 1)
    qm = jnp.where((i0 % 8) == (i1 // _PAD_C), q, 0.0)
    o_ref[...] = qm.reshape(r8 // 8, 8, 128).sum(axis=1) + b_ref[...]


def _project_table(emb_t, w_tiled, b_tiled):
    e, v = emb_t.shape
    grid = (v + _ROW_BLK - 1) // _ROW_BLK
    return pl.pallas_call(
        _proj_body,
        grid=(grid,),
        in_specs=[
            pl.BlockSpec((e, _ROW_BLK), lambda i: (0, i)),
            pl.BlockSpec((e, 128), lambda i: (0, 0)),
            pl.BlockSpec((1, 128), lambda i: (0, 0)),
        ],
        out_specs=pl.BlockSpec((_ROW_BLK // 8, 128), lambda i: (i, 0)),
        out_shape=jax.ShapeDtypeStruct((v // 8, 128), jnp.float32),
    )(emb_t, w_tiled, b_tiled)


def _make_sc_pool(n_batch, seq_len, n_workers):
    per_w_tok = n_batch * seq_len // n_workers
    per_w_b = n_batch // n_workers
    chunk = 128
    n_chunks = per_w_tok // chunk
    fire = 10
    mesh = plsc.VectorSubcoreMesh(core_axis_name="c", subcore_axis_name="s")

    @functools.partial(
        pl.kernel,
        out_type=jax.ShapeDtypeStruct((n_batch, _PAD_C), jnp.float32),
        mesh=mesh,
        scratch_types=[
            pltpu.VMEM((n_chunks, chunk), jnp.int32),
            pltpu.VMEM((per_w_tok, _PAD_C), jnp.float32),
            pltpu.VMEM((per_w_b, _PAD_C), jnp.float32),
            pltpu.SemaphoreType.DMA,
        ],
        compiler_params=pltpu.CompilerParams(use_tc_tiling_on_sc=False),
    )
    def sc_pool(idx_hbm, p_hbm, out_hbm, idx_v, rows_v, out_v, sem):
        nc = mesh.num_cores
        wid = lax.axis_index("s") * nc + lax.axis_index("c")

        pltpu.sync_copy(idx_hbm.at[wid], idx_v)

        def gather_group(g, _):
            base = g * fire
            copies = []
            for i in range(fire):
                j = base + i
                copies.append(
                    pltpu.async_copy(
                        p_hbm.at[idx_v.at[j]],
                        rows_v.at[pl.ds(j * chunk, chunk)],
                        sem,
                    )
                )
            for c in copies:
                c.wait()
            return _

        lax.fori_loop(0, n_chunks // fire, gather_group, None)

        def pool_one(b, _):
            base = b * seq_len
            acc0 = rows_v[base]
            acc1 = rows_v[base + 1]
            for t in range(2, seq_len, 2):
                acc0 = acc0 + rows_v[base + t]
                acc1 = acc1 + rows_v[base + t + 1]
            out_v[b] = acc0 + acc1
            return _

        lax.fori_loop(0, per_w_b, pool_one, None)

        pltpu.sync_copy(out_v, out_hbm.at[pl.ds(wid * per_w_b, per_w_b)])

    return sc_pool


def kernel(text, emb_table, fc_w, fc_b):
    n_batch, seq_len = text.shape
    v, e = emb_table.shape
    c = fc_w.shape[1]
    info = plsc.get_sparse_core_info()
    n_workers = info.num_cores * info.num_subcores

    inv_l = 1.0 / seq_len
    w_pad = jnp.zeros((e, _PAD_C), jnp.float32).at[:, :c].set(fc_w * inv_l)
    b_pad = jnp.zeros((1, _PAD_C), jnp.float32).at[0, :c].set(fc_b * inv_l)
    w_tiled = jnp.tile(w_pad, (1, 128 // _PAD_C))
    b_tiled = jnp.tile(b_pad, (1, 128 // _PAD_C))
    proj = _project_table(emb_table.T, w_tiled, b_tiled)
    proj16 = proj.reshape(v, _PAD_C)

    idx = text.astype(jnp.int32).reshape(n_workers, -1, 128)
    pooled = _make_sc_pool(n_batch, seq_len, n_workers)(idx, proj16)
    return pooled[:, :c]


# speedup vs baseline: 1.1183x; 1.0416x over previous
import functools

import jax
import jax.numpy as jnp
from jax import lax
from jax.experimental import pallas as pl
from jax.experimental.pallas import tpu as pltpu, tpu_sc as plsc

_PAD_C = 16
_ROW_BLK = 8192


def _proj_body(e_ref, w_ref, b_ref, o_ref):
    _, r8 = e_ref.shape
    q = lax.dot_general(
        e_ref[...], w_ref[...],
        (((0,), (0,)), ((), ())),
        preferred_element_type=jnp.float32,
    )
    i0 = lax.broadcasted_iota(jnp.int32, q.shape, 0)
    i1 = lax.broadcasted_iota(jnp.int32, q.shape, 1)
    qm = jnp.where((i0 % 8) == (i1 // _PAD_C), q, 0.0)
    o_ref[...] = qm.reshape(r8 // 8, 8, 128).sum(axis=1) + b_ref[...]


def _project_table(emb_t, w_tiled, b_tiled):
    e, v = emb_t.shape
    grid = (v + _ROW_BLK - 1) // _ROW_BLK
    return pl.pallas_call(
        _proj_body,
        grid=(grid,),
        in_specs=[
            pl.BlockSpec((e, _ROW_BLK), lambda i: (0, i)),
            pl.BlockSpec((e, 128), lambda i: (0, 0)),
            pl.BlockSpec((1, 128), lambda i: (0, 0)),
        ],
        out_specs=pl.BlockSpec((_ROW_BLK // 8, 128), lambda i: (i, 0)),
        out_shape=jax.ShapeDtypeStruct((v // 8, 128), jnp.float32),
    )(emb_t, w_tiled, b_tiled)


def _make_sc_pool(n_batch, seq_len, n_workers):
    per_w_tok = n_batch * seq_len // n_workers
    per_w_b = n_batch // n_workers
    chunk = 128
    n_chunks = per_w_tok // chunk
    fire = 10
    mesh = plsc.VectorSubcoreMesh(core_axis_name="c", subcore_axis_name="s")

    @functools.partial(
        pl.kernel,
        out_type=jax.ShapeDtypeStruct((n_batch, _PAD_C), jnp.float32),
        mesh=mesh,
        scratch_types=[
            pltpu.VMEM((n_chunks, chunk), jnp.int32),
            pltpu.VMEM((per_w_tok, _PAD_C), jnp.float32),
            pltpu.VMEM((per_w_b, _PAD_C), jnp.float32),
            pltpu.SemaphoreType.DMA,
            pltpu.SemaphoreType.DMA,
        ],
        compiler_params=pltpu.CompilerParams(use_tc_tiling_on_sc=False),
    )
    def sc_pool(idx_hbm, p_hbm, out_hbm, idx_v, rows_v, out_v, sem_a, sem_b):
        nc = mesh.num_cores
        wid = lax.axis_index("s") * nc + lax.axis_index("c")
        g_chunks = n_chunks // 2
        g_rows = per_w_b // 2

        pltpu.sync_copy(idx_hbm.at[wid], idx_v)

        def fire(g, sem):
            def body(j, _):
                k = g * g_chunks + j
                pltpu.async_copy(
                    p_hbm.at[idx_v.at[k]],
                    rows_v.at[pl.ds(k * chunk, chunk)],
                    sem,
                )
                return _
            lax.fori_loop(0, g_chunks, body, None)

        def drain(g, sem):
            def body(j, _):
                k = g * g_chunks + j
                pltpu.make_async_copy(
                    p_hbm.at[idx_v.at[k]],
                    rows_v.at[pl.ds(k * chunk, chunk)],
                    sem,
                ).wait()
                return _
            lax.fori_loop(0, g_chunks, body, None)

        def pool(g):
            def pool_one(r, _):
                b = g * g_rows + r
                base = b * seq_len
                acc0 = rows_v[base]
                acc1 = rows_v[base + 1]
                for t in range(2, seq_len, 2):
                    acc0 = acc0 + rows_v[base + t]
                    acc1 = acc1 + rows_v[base + t + 1]
                out_v[b] = acc0 + acc1
                return _
            lax.fori_loop(0, g_rows, pool_one, None)

        fire(0, sem_a)
        fire(1, sem_b)
        drain(0, sem_a)
        pool(0)
        drain(1, sem_b)
        pool(1)

        pltpu.sync_copy(out_v, out_hbm.at[pl.ds(wid * per_w_b, per_w_b)])

    return sc_pool


def kernel(text, emb_table, fc_w, fc_b):
    n_batch, seq_len = text.shape
    v, e = emb_table.shape
    c = fc_w.shape[1]
    info = plsc.get_sparse_core_info()
    n_workers = info.num_cores * info.num_subcores

    inv_l = 1.0 / seq_len
    w_pad = jnp.zeros((e, _PAD_C), jnp.float32).at[:, :c].set(fc_w * inv_l)
    b_pad = jnp.zeros((1, _PAD_C), jnp.float32).at[0, :c].set(fc_b * inv_l)
    w_tiled = jnp.tile(w_pad, (1, 128 // _PAD_C))
    b_tiled = jnp.tile(b_pad, (1, 128 // _PAD_C))
    proj = _project_table(emb_table.T, w_tiled, b_tiled)
    proj16 = proj.reshape(v, _PAD_C)

    idx = text.astype(jnp.int32).reshape(n_workers, -1, 128)
    pooled = _make_sc_pool(n_batch, seq_len, n_workers)(idx, proj16)
    return pooled[:, :c]


# speedup vs baseline: 1.1680x; 1.0444x over previous
import functools

import jax
import jax.numpy as jnp
from jax import lax
from jax.experimental import pallas as pl
from jax.experimental.pallas import tpu as pltpu, tpu_sc as plsc

_PAD_C = 16
_ROW_BLK = 8192


def _proj_body(inv_l, e_ref, wt_ref, b_ref, o_ref):
    _, r8 = e_ref.shape
    c, e = wt_ref.shape
    w = wt_ref[...].T * inv_l
    w_pad = jnp.concatenate(
        [w, jnp.zeros((e, _PAD_C - c), jnp.float32)], axis=1)
    w_tiled = jnp.tile(w_pad, (1, 128 // _PAD_C))
    b = b_ref[...] * inv_l
    b_pad = jnp.concatenate(
        [b, jnp.zeros((1, _PAD_C - c), jnp.float32)], axis=1)
    b_tiled = jnp.tile(b_pad, (1, 128 // _PAD_C))

    q = lax.dot_general(
        e_ref[...], w_tiled,
        (((0,), (0,)), ((), ())),
        preferred_element_type=jnp.float32,
    )
    i0 = lax.broadcasted_iota(jnp.int32, q.shape, 0)
    i1 = lax.broadcasted_iota(jnp.int32, q.shape, 1)
    qm = jnp.where((i0 % 8) == (i1 // _PAD_C), q, 0.0)
    o_ref[...] = qm.reshape(r8 // 8, 8, 128).sum(axis=1) + b_tiled


def _project_table(emb_t, fc_w_t, fc_b_row, inv_l):
    e, v = emb_t.shape
    c = fc_w_t.shape[0]
    grid = (v + _ROW_BLK - 1) // _ROW_BLK
    return pl.pallas_call(
        functools.partial(_proj_body, inv_l),
        grid=(grid,),
        in_specs=[
            pl.BlockSpec((e, _ROW_BLK), lambda i: (0, i)),
            pl.BlockSpec((c, e), lambda i: (0, 0)),
            pl.BlockSpec((1, c), lambda i: (0, 0)),
        ],
        out_specs=pl.BlockSpec((_ROW_BLK // 8, 128), lambda i: (i, 0)),
        out_shape=jax.ShapeDtypeStruct((v // 8, 128), jnp.float32),
    )(emb_t, fc_w_t, fc_b_row)


def _make_sc_pool(n_batch, seq_len, n_workers):
    per_w_tok = n_batch * seq_len // n_workers
    per_w_b = n_batch // n_workers
    chunk = 128
    n_chunks = per_w_tok // chunk
    fire = 10
    mesh = plsc.VectorSubcoreMesh(core_axis_name="c", subcore_axis_name="s")

    @functools.partial(
        pl.kernel,
        out_type=jax.ShapeDtypeStruct((n_batch, _PAD_C), jnp.float32),
        mesh=mesh,
        scratch_types=[
            pltpu.VMEM((n_chunks, chunk), jnp.int32),
            pltpu.VMEM((per_w_tok, _PAD_C), jnp.float32),
            pltpu.VMEM((per_w_b, _PAD_C), jnp.float32),
            pltpu.SemaphoreType.DMA,
            pltpu.SemaphoreType.DMA,
        ],
        compiler_params=pltpu.CompilerParams(use_tc_tiling_on_sc=False),
    )
    def sc_pool(idx_hbm, p_hbm, out_hbm, idx_v, rows_v, out_v, sem_a, sem_b):
        nc = mesh.num_cores
        wid = lax.axis_index("s") * nc + lax.axis_index("c")
        g_chunks = n_chunks // 2
        g_rows = per_w_b // 2

        pltpu.sync_copy(idx_hbm.at[wid], idx_v)

        def fire(g, sem):
            def body(j, _):
                k = g * g_chunks + j
                pltpu.async_copy(
                    p_hbm.at[idx_v.at[k]],
                    rows_v.at[pl.ds(k * chunk, chunk)],
                    sem,
                )
                return _
            lax.fori_loop(0, g_chunks, body, None)

        def drain(g, sem):
            def body(j, _):
                k = g * g_chunks + j
                pltpu.make_async_copy(
                    p_hbm.at[idx_v.at[k]],
                    rows_v.at[pl.ds(k * chunk, chunk)],
                    sem,
                ).wait()
                return _
            lax.fori_loop(0, g_chunks, body, None)

        def pool(g):
            def pool_one(r, _):
                b = g * g_rows + r
                base = b * seq_len
                acc0 = rows_v[base]
                acc1 = rows_v[base + 1]
                for t in range(2, seq_len, 2):
                    acc0 = acc0 + rows_v[base + t]
                    acc1 = acc1 + rows_v[base + t + 1]
                out_v[b] = acc0 + acc1
                return _
            lax.fori_loop(0, g_rows, pool_one, None)

        fire(0, sem_a)
        fire(1, sem_b)
        drain(0, sem_a)
        pool(0)
        drain(1, sem_b)
        pool(1)

        pltpu.sync_copy(out_v, out_hbm.at[pl.ds(wid * per_w_b, per_w_b)])

    return sc_pool


def kernel(text, emb_table, fc_w, fc_b):
    n_batch, seq_len = text.shape
    v, e = emb_table.shape
    c = fc_w.shape[1]
    info = plsc.get_sparse_core_info()
    n_workers = info.num_cores * info.num_subcores

    inv_l = 1.0 / seq_len
    proj = _project_table(emb_table.T, fc_w.T, fc_b[None, :], inv_l)
    proj16 = proj.reshape(v, _PAD_C)

    idx = text.astype(jnp.int32).reshape(n_workers, -1, 128)
    pooled = _make_sc_pool(n_batch, seq_len, n_workers)(idx, proj16)
    return pooled[:, :c]


# speedup vs baseline: 1.1736x; 1.0048x over previous
import functools

import jax
import jax.numpy as jnp
from jax import lax
from jax.experimental import pallas as pl
from jax.experimental.pallas import tpu as pltpu, tpu_sc as plsc

_PAD_C = 16
_ROW_BLK = 8192


def _proj_body(inv_l, e_ref, wt_ref, b_ref, o_ref):
    _, r8 = e_ref.shape
    c, e = wt_ref.shape
    w = wt_ref[...].T * inv_l
    w_pad = jnp.concatenate(
        [w, jnp.zeros((e, _PAD_C - c), jnp.float32)], axis=1)
    w_tiled = jnp.tile(w_pad, (1, 128 // _PAD_C))
    b = b_ref[...] * inv_l
    b_pad = jnp.concatenate(
        [b, jnp.zeros((1, _PAD_C - c), jnp.float32)], axis=1)
    b_tiled = jnp.tile(b_pad, (1, 128 // _PAD_C))

    q = lax.dot_general(
        e_ref[...], w_tiled,
        (((0,), (0,)), ((), ())),
        preferred_element_type=jnp.float32,
    )
    i0 = lax.broadcasted_iota(jnp.int32, q.shape, 0)
    i1 = lax.broadcasted_iota(jnp.int32, q.shape, 1)
    qm = jnp.where((i0 % 8) == (i1 // _PAD_C), q, 0.0)
    o_ref[...] = qm.reshape(r8 // 8, 8, 128).sum(axis=1) + b_tiled


def _project_table(emb_t, fc_w_t, fc_b_row, inv_l):
    e, v = emb_t.shape
    c = fc_w_t.shape[0]
    grid = (v + _ROW_BLK - 1) // _ROW_BLK
    return pl.pallas_call(
        functools.partial(_proj_body, inv_l),
        grid=(grid,),
        in_specs=[
            pl.BlockSpec((e, _ROW_BLK), lambda i: (0, i)),
            pl.BlockSpec((c, e), lambda i: (0, 0)),
            pl.BlockSpec((1, c), lambda i: (0, 0)),
        ],
        out_specs=pl.BlockSpec((_ROW_BLK // 8, 128), lambda i: (i, 0)),
        out_shape=jax.ShapeDtypeStruct((v // 8, 128), jnp.float32),
    )(emb_t, fc_w_t, fc_b_row)


def _make_sc_pool(n_batch, seq_len, n_workers):
    per_w_tok = n_batch * seq_len // n_workers
    per_w_b = n_batch // n_workers
    chunk = 2 * seq_len
    n_chunks = per_w_tok // chunk
    n_groups = 4
    mesh = plsc.VectorSubcoreMesh(core_axis_name="c", subcore_axis_name="s")

    @functools.partial(
        pl.kernel,
        out_type=jax.ShapeDtypeStruct((n_batch, _PAD_C), jnp.float32),
        mesh=mesh,
        scratch_types=[
            pltpu.VMEM((n_chunks, chunk), jnp.int32),
            pltpu.VMEM((per_w_tok, _PAD_C), jnp.float32),
            pltpu.VMEM((per_w_b, _PAD_C), jnp.float32),
            pltpu.SemaphoreType.DMA,
            pltpu.SemaphoreType.DMA,
            pltpu.SemaphoreType.DMA,
            pltpu.SemaphoreType.DMA,
        ],
        compiler_params=pltpu.CompilerParams(use_tc_tiling_on_sc=False),
    )
    def sc_pool(idx_hbm, p_hbm, out_hbm, idx_v, rows_v, out_v, *sems):
        nc = mesh.num_cores
        wid = lax.axis_index("s") * nc + lax.axis_index("c")
        g_chunks = n_chunks // n_groups
        g_rows = per_w_b // n_groups

        pltpu.sync_copy(idx_hbm.at[wid], idx_v)

        def fire(g, sem):
            def body(j, _):
                k = g * g_chunks + j
                pltpu.async_copy(
                    p_hbm.at[idx_v.at[k]],
                    rows_v.at[pl.ds(k * chunk, chunk)],
                    sem,
                )
                return _
            lax.fori_loop(0, g_chunks, body, None)

        def drain(g, sem):
            def body(j, _):
                k = g * g_chunks + j
                pltpu.make_async_copy(
                    p_hbm.at[idx_v.at[k]],
                    rows_v.at[pl.ds(k * chunk, chunk)],
                    sem,
                ).wait()
                return _
            lax.fori_loop(0, g_chunks, body, None)

        def pool(g):
            def pool_one(r, _):
                b = g * g_rows + r
                base = b * seq_len
                acc0 = rows_v[base]
                acc1 = rows_v[base + 1]
                for t in range(2, seq_len, 2):
                    acc0 = acc0 + rows_v[base + t]
                    acc1 = acc1 + rows_v[base + t + 1]
                out_v[b] = acc0 + acc1
                return _
            lax.fori_loop(0, g_rows, pool_one, None)

        for g in range(n_groups):
            fire(g, sems[g])
        for g in range(n_groups):
            drain(g, sems[g])
            pool(g)

        pltpu.sync_copy(out_v, out_hbm.at[pl.ds(wid * per_w_b, per_w_b)])

    return sc_pool


def kernel(text, emb_table, fc_w, fc_b):
    n_batch, seq_len = text.shape
    v, e = emb_table.shape
    c = fc_w.shape[1]
    info = plsc.get_sparse_core_info()
    n_workers = info.num_cores * info.num_subcores

    inv_l = 1.0 / seq_len
    proj = _project_table(emb_table.T, fc_w.T, fc_b[None, :], inv_l)
    proj16 = proj.reshape(v, _PAD_C)

    idx = text.astype(jnp.int32).reshape(n_workers, -1, 2 * seq_len)
    pooled = _make_sc_pool(n_batch, seq_len, n_workers)(idx, proj16)
    return pooled[:, :c]
